# baseline (speedup 1.0000x reference)
import functools

import jax
import jax.numpy as jnp
from jax import lax
from jax.experimental import pallas as pl
from jax.experimental.pallas import tpu as pltpu
from jax.experimental.pallas import tpu_sc as plsc

F_N, L_N, E_N, V_N = 10000, 20000, 40000, 30000
EMB = 64
K = 6
C = 512
NSUB = 16
RB = 40
TCB = 1000


_I16 = None


def _scalar_popcount(mask):
    r = plsc.all_reduce_population_count(mask)
    if getattr(r, "ndim", 0):
        r = jnp.max(r)
    return r


@functools.cache
def _make_partition(n_pad, H):
    NQ = n_pad // C
    P = n_pad // 32
    NCI = P // C
    mesh = plsc.VectorSubcoreMesh(core_axis_name="c", subcore_axis_name="s")

    @functools.partial(
        pl.kernel,
        out_type=(jax.ShapeDtypeStruct((2, n_pad), jnp.int32),
                  jax.ShapeDtypeStruct((2, n_pad), jnp.int32),
                  jax.ShapeDtypeStruct((32, 16), jnp.int32)),
        mesh=mesh,
        compiler_params=pltpu.CompilerParams(use_tc_tiling_on_sc=False,
                                             needs_layout_passes=False),
        scratch_types=[
            pltpu.VMEM((C,), jnp.int32), pltpu.VMEM((C,), jnp.int32),
            pltpu.VMEM((P + 16,), jnp.int32), pltpu.VMEM((P + 16,), jnp.int32),
            pltpu.VMEM((P + 16,), jnp.int32), pltpu.VMEM((P + 16,), jnp.int32),
            pltpu.VMEM((16,), jnp.int32),
        ],
    )
    def part(src_in, dst_in, srcp, dstp, counts, cbs, cbd,
             ls0, ld0, ls1, ld1, cnt):
        c = lax.axis_index("c")
        s = lax.axis_index("s")
        w = s * 2 + c
        zi = jnp.zeros((16,), jnp.int32)
        hi = jnp.full((16,), H, jnp.int32)

        def fbody(j, carry):
            ls0[pl.ds(j * 16, 16)] = zi
            ls1[pl.ds(j * 16, 16)] = zi
            ld0[pl.ds(j * 16, 16)] = hi
            ld1[pl.ds(j * 16, 16)] = hi
            return carry
        lax.fori_loop(0, (P + 16) // 16, fbody, 0)

        one = jnp.ones((16,), jnp.int32)
        zero = jnp.zeros((16,), jnp.int32)

        def chunk(q, offs):
            off0, off1 = offs
            pltpu.sync_copy(src_in.at[w * NCI + q], cbs)
            pltpu.sync_copy(dst_in.at[w * NCI + q], cbd)
            for j in range(C // 16):
                sv = cbs[pl.ds(j * 16, 16)]
                dv = cbd[pl.ds(j * 16, 16)]
                m0 = (dv >= 0) & (dv < H)
                m1 = dv >= H
                key0 = jnp.where(m0, zero, one)
                key1 = jnp.where(m1, zero, one)
                ls0[pl.ds(off0, 16)] = plsc.sort_key_val(key0, sv)[1]
                ld0[pl.ds(off0, 16)] = plsc.sort_key_val(key0, dv)[1]
                ls1[pl.ds(off1, 16)] = plsc.sort_key_val(key1, sv)[1]
                ld1[pl.ds(off1, 16)] = plsc.sort_key_val(key1, dv - H)[1]
                off0 = off0 + jnp.sum(jnp.where(m0, one, zero))
                off1 = off1 + jnp.sum(jnp.where(m1, one, zero))
            return (off0, off1)
        k0, k1 = lax.fori_loop(0, NCI, chunk, (jnp.int32(0), jnp.int32(0)))
        ls0[pl.ds(k0, 16)] = zi
        ld0[pl.ds(k0, 16)] = hi
        ls1[pl.ds(k1, 16)] = zi
        ld1[pl.ds(k1, 16)] = hi

        lanes = lax.iota(jnp.int32, 16)
        cnt[...] = jnp.where(lanes == 0, k0, jnp.where(lanes == 1, k1, 0))
        pltpu.sync_copy(cnt, counts.at[w])
        pltpu.sync_copy(ls0.at[pl.ds(0, P)], srcp.at[0, pl.ds(w * P, P)])
        pltpu.sync_copy(ld0.at[pl.ds(0, P)], dstp.at[0, pl.ds(w * P, P)])
        pltpu.sync_copy(ls1.at[pl.ds(0, P)], srcp.at[1, pl.ds(w * P, P)])
        pltpu.sync_copy(ld1.at[pl.ds(0, P)], dstp.at[1, pl.ds(w * P, P)])

    return part


@functools.cache
def _make_seg_sum(N_src, N_dst, n_pad):
    H = N_dst // 2
    A = H + RB
    P = n_pad // 32
    NCS = P // C
    NZ = A // RB
    NW = H // RB
    mesh = plsc.VectorSubcoreMesh(core_axis_name="c", subcore_axis_name="s")

    @functools.partial(
        pl.kernel,
        out_type=jax.ShapeDtypeStruct((2, A, EMB), jnp.float32),
        mesh=mesh,
        compiler_params=pltpu.CompilerParams(use_tc_tiling_on_sc=False,
                                             needs_layout_passes=False),
        scratch_types=[
            pltpu.VMEM_SHARED((A, EMB), jnp.float32),
            pltpu.VMEM((C,), jnp.int32),
            pltpu.VMEM((C,), jnp.int32),
            pltpu.VMEM((C, EMB), jnp.float32),
            pltpu.VMEM((RB, EMB), jnp.float32),
            pltpu.VMEM((16,), jnp.int32),
            pltpu.SemaphoreType.DMA,
            pltpu.SemaphoreType.DMA,
            pltpu.SemaphoreType.DMA,
        ],
    )
    def seg_sum(x_hbm, srcp_hbm, dstp_hbm, counts_hbm, out_hbm,
                acc, ib3, db3, rows, zb, cnt, gsem, ssem, hsem):
        c = lax.axis_index("c")
        s = lax.axis_index("s")

        z16 = jnp.zeros((16,), jnp.float32)
        for r in range(RB):
            for q in range(EMB // 16):
                zb[r, pl.ds(q * 16, 16)] = z16

        nz_mine = NZ // NSUB + (1 if NZ % NSUB else 0)

        def zbody(j, carry):
            cid = j * NSUB + s
            @pl.when(cid < NZ)
            def _():
                pltpu.async_copy(zb, acc.at[pl.ds(cid * RB, RB)], hsem)
            return carry
        lax.fori_loop(0, nz_mine, zbody, 0)

        def zdrain(j, carry):
            cid = j * NSUB + s
            @pl.when(cid < NZ)
            def _():
                pltpu.make_async_copy(zb, acc.at[pl.ds(0, RB)], hsem).wait()
            return carry
        lax.fori_loop(0, nz_mine, zdrain, 0)

        plsc.subcore_barrier()

        lanes = lax.iota(jnp.int32, 16)
        for h in range(2):
            sec = s * 2 + h
            pltpu.sync_copy(counts_hbm.at[sec], cnt)
            k = jnp.max(jnp.where(lanes == c, cnt[...], 0))
            nch = (k + C - 1) // C

            def body(q, carry):
                off = sec * P + q * C
                pltpu.sync_copy(srcp_hbm.at[c, pl.ds(off, C)], ib3)
                pltpu.sync_copy(dstp_hbm.at[c, pl.ds(off, C)], db3)
                pltpu.sync_copy(x_hbm.at[ib3], rows)
                pltpu.sync_copy(rows, acc.at[db3], add=True)
                return carry
            lax.fori_loop(0, nch, body, 0)
        plsc.subcore_barrier()

        nw_mine = NW // NSUB + (1 if NW % NSUB else 0)

        def wbody(j, carry):
            cid = j * NSUB + s
            @pl.when(cid < NW)
            def _():
                pltpu.async_copy(acc.at[pl.ds(cid * RB, RB)],
                                 out_hbm.at[c, pl.ds(cid * RB, RB)], hsem)
            return carry
        lax.fori_loop(0, nw_mine, wbody, 0)

        def wdrain(j, carry):
            cid = j * NSUB + s
            @pl.when(cid < NW)
            def _():
                pltpu.make_async_copy(
                    acc.at[pl.ds(0, RB)],
                    out_hbm.at[c, pl.ds(0, RB)], hsem).wait()
            return carry
        lax.fori_loop(0, nw_mine, wdrain, 0)

    return seg_sum


def _prep_dir(src_idx, dst_idx, N_dst):
    n = src_idx.shape[0]
    H = N_dst // 2
    n_pad = -(-n // (32 * C)) * (32 * C)
    pad = n_pad - n
    src_p = jnp.pad(src_idx, (0, pad))
    dst_p = jnp.pad(dst_idx, (0, pad), constant_values=-1)
    NQ = n_pad // C
    srcp, dstp, counts = _make_partition(n_pad, H)(
        src_p.reshape(NQ, C), dst_p.reshape(NQ, C))
    return (srcp, dstp, counts, n_pad)


def _seg_sum(x, srcp, dstp, counts, n_pad, N_dst):
    return _make_seg_sum(x.shape[0], N_dst, n_pad)(x, srcp, dstp, counts)


def _embed_body(x_ref, w_ref, b_ref, o_ref):
    o_ref[...] = jnp.maximum(
        jnp.dot(x_ref[...], w_ref[...], preferred_element_type=jnp.float32)
        + b_ref[...], 0.0)


@functools.cache
def _make_embed(N, S):
    return pl.pallas_call(
        _embed_body,
        grid=(N // TCB,),
        in_specs=[pl.BlockSpec((TCB, S), lambda i: (i, 0)),
                  pl.BlockSpec((S, EMB), lambda i: (0, 0)),
                  pl.BlockSpec((1, EMB), lambda i: (0, 0))],
        out_specs=pl.BlockSpec((TCB, EMB), lambda i: (i, 0)),
        out_shape=jax.ShapeDtypeStruct((N, EMB), jnp.float32),
    )


def _embed(x, w, b):
    return _make_embed(x.shape[0], x.shape[1])(x, w, b.reshape(1, EMB))


def _stage_body(d_ref, s_ref, w_ref, o_ref):
    o_ref[...] = jnp.maximum(
        d_ref[...] + jnp.dot(s_ref[0], w_ref[...],
                             preferred_element_type=jnp.float32), 0.0)


@functools.cache
def _make_stage(N, A):
    HB = (N // 2) // TCB
    return pl.pallas_call(
        _stage_body,
        grid=(N // TCB,),
        in_specs=[pl.BlockSpec((TCB, EMB), lambda i: (i, 0)),
                  pl.BlockSpec((1, TCB, EMB), lambda i: (i // HB, i % HB, 0)),
                  pl.BlockSpec((EMB, EMB), lambda i: (0, 0))],
        out_specs=pl.BlockSpec((TCB, EMB), lambda i: (i, 0)),
        out_shape=jax.ShapeDtypeStruct((N, EMB), jnp.float32),
    )


def _stage(dst, x, part, W):
    N_dst = dst.shape[0]
    s2 = _seg_sum(x, *part, N_dst)
    return _make_stage(N_dst, s2.shape[1])(dst, s2, W)


def kernel(left_faces, left_loops, left_edges, left_verts,
           right_faces, right_loops, right_edges, right_verts,
           left_face_to_loop, left_loop_to_edge, left_edge_to_vertex,
           left_face_to_face, right_face_to_loop, right_loop_to_edge,
           right_edge_to_vertex, right_face_to_face,
           Wf, bf, Wl, bl, We, be, Wv, bv,
           W_ve, W_el, W_lf, W_ff, W_fl, W_le, W_ev):
    def side(faces, loops, edges, verts, f2l, l2e, e2v, f2f):
        f = _embed(faces, Wf, bf)
        l = _embed(loops, Wl, bl)
        e = _embed(edges, We, be)
        v = _embed(verts, Wv, bv)
        up_ve = _prep_dir(e2v[1], e2v[0], E_N)
        up_el = _prep_dir(l2e[1], l2e[0], L_N)
        up_lf = _prep_dir(f2l[1], f2l[0], F_N)
        up_ff = _prep_dir(f2f[1], f2f[0], F_N)
        dn_fl = _prep_dir(f2l[0], f2l[1], L_N)
        dn_le = _prep_dir(l2e[0], l2e[1], E_N)
        dn_ev = _prep_dir(e2v[0], e2v[1], V_N)
        for _ in range(K):
            e = _stage(e, v, up_ve, W_ve)
            l = _stage(l, e, up_el, W_el)
            f = _stage(f, l, up_lf, W_lf)
            f = _stage(f, f, up_ff, W_ff)
            l = _stage(l, f, dn_fl, W_fl)
            e = _stage(e, l, dn_le, W_le)
            v = _stage(v, e, dn_ev, W_ev)
        return f, e, v

    out_l = side(left_faces, left_loops, left_edges, left_verts,
                 left_face_to_loop, left_loop_to_edge, left_edge_to_vertex,
                 left_face_to_face)
    out_r = side(right_faces, right_loops, right_edges, right_verts,
                 right_face_to_loop, right_loop_to_edge, right_edge_to_vertex,
                 right_face_to_face)
    return (out_l, out_r)


# speedup vs baseline: 2.8007x; 2.8007x over previous
import functools

import jax
import jax.numpy as jnp
from jax import lax
from jax.experimental import pallas as pl
from jax.experimental.pallas import tpu as pltpu
from jax.experimental.pallas import tpu_sc as plsc

F_N, L_N, E_N, V_N = 10000, 20000, 40000, 30000
EMB = 64
K = 6
C = 128
NSUB = 16
RB = 40
TCB = 1000


@functools.cache
def _make_seg_sum(N_src, N_dst, NQ):
    H = N_dst // 2
    A = H + RB
    NCH = NQ // NSUB
    NZ = A // RB
    NW = H // RB
    mesh = plsc.VectorSubcoreMesh(core_axis_name="c", subcore_axis_name="s")

    @functools.partial(
        pl.kernel,
        out_type=jax.ShapeDtypeStruct((2, A, EMB), jnp.float32),
        mesh=mesh,
        compiler_params=pltpu.CompilerParams(use_tc_tiling_on_sc=False),
        scratch_types=[
            pltpu.VMEM_SHARED((A, EMB), jnp.float32),
            pltpu.VMEM((C,), jnp.int32), pltpu.VMEM((C,), jnp.int32),
            pltpu.VMEM((C,), jnp.int32), pltpu.VMEM((C,), jnp.int32),
            pltpu.VMEM((C, EMB), jnp.float32), pltpu.VMEM((C, EMB), jnp.float32),
            pltpu.VMEM((RB, EMB), jnp.float32),
            pltpu.SemaphoreType.DMA, pltpu.SemaphoreType.DMA,
        ],
    )
    def seg_sum(x_hbm, src_hbm, dst_hbm, out_hbm,
                acc, ib0, ib1, db0, db1, rb0, rb1, zb, sem0, sem1):
        c = lax.axis_index("c")
        s = lax.axis_index("s")
        ibs, dbs, rbs, sems = (ib0, ib1), (db0, db1), (rb0, rb1), (sem0, sem1)

        z16 = jnp.zeros((16,), jnp.float32)
        for r in range(RB):
            for q in range(EMB // 16):
                zb[r, pl.ds(q * 16, 16)] = z16

        def zbody(j, carry):
            cid = j * NSUB + s
            @pl.when(cid < NZ)
            def _():
                pltpu.sync_copy(zb, acc.at[pl.ds(cid * RB, RB)])
            return carry
        lax.fori_loop(0, (NZ + NSUB - 1) // NSUB, zbody, 0)
        plsc.subcore_barrier()

        q0 = s * NCH
        pltpu.sync_copy(src_hbm.at[q0], ib0)
        pltpu.sync_copy(dst_hbm.at[c, q0], db0)
        pltpu.async_copy(x_hbm.at[ib0], rb0, sem0)

        def body(j, carry):
            for b in range(2):
                jj = j * 2 + b
                cur, nxt = b, 1 - b
                @pl.when(jj + 1 < NCH)
                def _():
                    qn = s * NCH + jj + 1
                    pltpu.sync_copy(src_hbm.at[qn], ibs[nxt])
                    pltpu.sync_copy(dst_hbm.at[c, qn], dbs[nxt])
                    pltpu.async_copy(x_hbm.at[ibs[nxt]], rbs[nxt], sems[nxt])
                pltpu.make_async_copy(x_hbm.at[ibs[cur]], rbs[cur],
                                      sems[cur]).wait()
                pltpu.sync_copy(rbs[cur], acc.at[dbs[cur]], add=True)
            return carry
        lax.fori_loop(0, NCH // 2, body, 0)
        plsc.subcore_barrier()

        def wbody(j, carry):
            cid = j * NSUB + s
            @pl.when(cid < NW)
            def _():
                pltpu.sync_copy(acc.at[pl.ds(cid * RB, RB)], zb)
                pltpu.sync_copy(zb, out_hbm.at[c, pl.ds(cid * RB, RB)])
            return carry
        lax.fori_loop(0, (NW + NSUB - 1) // NSUB, wbody, 0)

    return seg_sum


def _prep_dir(src_idx, dst_idx, N_dst):
    n = src_idx.shape[0]
    H = N_dst // 2
    n_pad = -(-n // (NSUB * C * 2)) * (NSUB * C * 2)
    pad = n_pad - n
    src_p = jnp.pad(src_idx, (0, pad))
    dst_p = jnp.pad(dst_idx, (0, pad), constant_values=-1)
    own0 = (dst_p >= 0) & (dst_p < H)
    own1 = dst_p >= H
    d0 = jnp.where(own0, dst_p, H)
    d1 = jnp.where(own1, dst_p - H, H)
    NQ = n_pad // C
    return (src_p.reshape(NQ, C),
            jnp.stack([d0, d1]).reshape(2, NQ, C).astype(jnp.int32), NQ)


def _seg_sum(x, src2d, dst3d, NQ, N_dst):
    return _make_seg_sum(x.shape[0], N_dst, NQ)(x, src2d, dst3d)


def _embed_body(x_ref, w_ref, b_ref, o_ref):
    o_ref[...] = jnp.maximum(
        jnp.dot(x_ref[...], w_ref[...], preferred_element_type=jnp.float32)
        + b_ref[...], 0.0)


@functools.cache
def _make_embed(N, S):
    return pl.pallas_call(
        _embed_body,
        grid=(N // TCB,),
        in_specs=[pl.BlockSpec((TCB, S), lambda i: (i, 0)),
                  pl.BlockSpec((S, EMB), lambda i: (0, 0)),
                  pl.BlockSpec((1, EMB), lambda i: (0, 0))],
        out_specs=pl.BlockSpec((TCB, EMB), lambda i: (i, 0)),
        out_shape=jax.ShapeDtypeStruct((N, EMB), jnp.float32),
    )


def _embed(x, w, b):
    return _make_embed(x.shape[0], x.shape[1])(x, w, b.reshape(1, EMB))


def _stage_body(d_ref, s_ref, w_ref, o_ref):
    o_ref[...] = jnp.maximum(
        d_ref[...] + jnp.dot(s_ref[0], w_ref[...],
                             preferred_element_type=jnp.float32), 0.0)


@functools.cache
def _make_stage(N, A):
    HB = (N // 2) // TCB
    return pl.pallas_call(
        _stage_body,
        grid=(N // TCB,),
        in_specs=[pl.BlockSpec((TCB, EMB), lambda i: (i, 0)),
                  pl.BlockSpec((1, TCB, EMB), lambda i: (i // HB, i % HB, 0)),
                  pl.BlockSpec((EMB, EMB), lambda i: (0, 0))],
        out_specs=pl.BlockSpec((TCB, EMB), lambda i: (i, 0)),
        out_shape=jax.ShapeDtypeStruct((N, EMB), jnp.float32),
    )


def _stage(dst, x, src2d, dst3d, NQ, W):
    N_dst = dst.shape[0]
    s2 = _seg_sum(x, src2d, dst3d, NQ, N_dst)
    return _make_stage(N_dst, s2.shape[1])(dst, s2, W)


def kernel(left_faces, left_loops, left_edges, left_verts,
           right_faces, right_loops, right_edges, right_verts,
           left_face_to_loop, left_loop_to_edge, left_edge_to_vertex,
           left_face_to_face, right_face_to_loop, right_loop_to_edge,
           right_edge_to_vertex, right_face_to_face,
           Wf, bf, Wl, bl, We, be, Wv, bv,
           W_ve, W_el, W_lf, W_ff, W_fl, W_le, W_ev):
    def side(faces, loops, edges, verts, f2l, l2e, e2v, f2f):
        f = _embed(faces, Wf, bf)
        l = _embed(loops, Wl, bl)
        e = _embed(edges, We, be)
        v = _embed(verts, Wv, bv)
        up_ve = _prep_dir(e2v[1], e2v[0], E_N)
        up_el = _prep_dir(l2e[1], l2e[0], L_N)
        up_lf = _prep_dir(f2l[1], f2l[0], F_N)
        up_ff = _prep_dir(f2f[1], f2f[0], F_N)
        dn_fl = _prep_dir(f2l[0], f2l[1], L_N)
        dn_le = _prep_dir(l2e[0], l2e[1], E_N)
        dn_ev = _prep_dir(e2v[0], e2v[1], V_N)
        for _ in range(K):
            e = _stage(e, v, *up_ve, W_ve)
            l = _stage(l, e, *up_el, W_el)
            f = _stage(f, l, *up_lf, W_lf)
            f = _stage(f, f, *up_ff, W_ff)
            l = _stage(l, f, *dn_fl, W_fl)
            e = _stage(e, l, *dn_le, W_le)
            v = _stage(v, e, *dn_ev, W_ev)
        return f, e, v

    out_l = side(left_faces, left_loops, left_edges, left_verts,
                 left_face_to_loop, left_loop_to_edge, left_edge_to_vertex,
                 left_face_to_face)
    out_r = side(right_faces, right_loops, right_edges, right_verts,
                 right_face_to_loop, right_loop_to_edge, right_edge_to_vertex,
                 right_face_to_face)
    return (out_l, out_r)
